# baseline (speedup 1.0000x reference)
import functools

import jax
import jax.numpy as jnp
from jax import lax
from jax.experimental import pallas as pl
from jax.experimental.pallas import tpu as pltpu
from jax.experimental.pallas import tpu_sc as plsc

NC = 2
NS = 16
NW = NC * NS

D = 64
G = 128
NBUF = 4
KJ = 16


def _sc_gather_body(ngroups, table_hbm, idx_hbm, out_hbm, idx_v, rows_v, gsem, osem):
    cid = lax.axis_index("c")
    sid = lax.axis_index("s")
    wid = sid * NC + cid
    base = wid * ngroups * G

    pltpu.sync_copy(idx_hbm.at[wid], idx_v)

    def start_gather(g, b):
        pltpu.async_copy(table_hbm.at[idx_v.at[g]], rows_v.at[b], gsem.at[b])

    def wait_gather(b):
        pltpu.make_async_copy(
            table_hbm.at[idx_v.at[0]], rows_v.at[b], gsem.at[b]
        ).wait()

    def start_out(g, b):
        pltpu.async_copy(
            rows_v.at[b], out_hbm.at[pl.ds(base + g * G, G)], osem.at[b]
        )

    def wait_out(b):
        pltpu.make_async_copy(
            rows_v.at[b], out_hbm.at[pl.ds(base, G)], osem.at[b]
        ).wait()

    def group(g, b, first, prefetch):
        pb = (b - 1) % NBUF
        if not first:
            wait_out(pb)
        if prefetch:
            start_gather(g + (NBUF - 1), pb)
        wait_gather(b)
        start_out(g, b)

    for b in range(NBUF - 1):
        start_gather(b, b)

    for b in range(NBUF):
        group(b, b, first=(b == 0), prefetch=True)

    @pl.loop(NBUF, ngroups - NBUF, step=NBUF)
    def steady(gbase):
        for b in range(NBUF):
            group(gbase + b, b, first=False, prefetch=True)

    for b in range(NBUF):
        group(ngroups - NBUF + b, b, first=False, prefetch=(b == 0))

    wait_out(NBUF - 1)


def _c_body(x_ref, o_ref):
    x = x_ref[...]

    def scale_of(h):
        ssq = jnp.sum(h * h, axis=1, keepdims=True)
        return jnp.minimum(
            jnp.float32(1.0), lax.rsqrt(jnp.maximum(ssq, jnp.float32(1e-14)))
        )

    e = x[:, 0:D]
    o = x[:, D : 2 * D]
    se = e * scale_of(e)
    so = o * scale_of(o)
    eye = jax.lax.broadcasted_iota(jnp.int32, (D, D), 0) == jax.lax.broadcasted_iota(
        jnp.int32, (D, D), 1
    )
    eye = eye.astype(jnp.float32)
    for j in range(KJ):
        blk = slice(j * D, (j + 1) * D)
        te = lax.dot_general(se[blk], eye, (((0,), (0,)), ((), ())))
        to = lax.dot_general(so[blk], eye, (((0,), (0,)), ((), ())))
        o_ref[0, :, j * G : (j + 1) * G] = jnp.concatenate([te, to], axis=1)


@functools.partial(jax.jit, static_argnames=())
def kernel(token_ids, table):
    S, P = token_ids.shape
    B = S * P
    assert B % (NW * G) == 0
    ngroups = B // (NW * G)
    assert ngroups % NBUF == 0 and ngroups >= 2 * NBUF

    u = jnp.arange(G, dtype=jnp.int32)
    perm = u // 2 + (u % 2) * (G // 2)
    idx = jnp.take(token_ids.T.reshape(P, S // G, G), perm, axis=2)
    idx = idx.reshape(NW, ngroups, G)

    mesh = plsc.VectorSubcoreMesh(
        core_axis_name="c", subcore_axis_name="s", num_cores=NC, num_subcores=NS
    )
    gathered = pl.kernel(
        functools.partial(_sc_gather_body, ngroups),
        out_type=jax.ShapeDtypeStruct((B, D), jnp.float32),
        mesh=mesh,
        scratch_types=[
            pltpu.VMEM((ngroups, G), jnp.int32),
            pltpu.VMEM((NBUF, G, D), jnp.float32),
            pltpu.SemaphoreType.DMA((NBUF,)),
            pltpu.SemaphoreType.DMA((NBUF,)),
        ],
        compiler_params=pltpu.CompilerParams(
            needs_layout_passes=False, use_tc_tiling_on_sc=False
        ),
    )(table, idx)

    c_in = gathered.reshape(-1).reshape(B // 2, 128)
    njb = S // (KJ * G)
    c_out = pl.pallas_call(
        _c_body,
        grid=(P, njb),
        in_specs=[pl.BlockSpec((KJ * D, G), lambda i, j: (i * njb + j, 0))],
        out_specs=pl.BlockSpec((1, D, KJ * G), lambda i, j: (i, 0, j)),
        out_shape=jax.ShapeDtypeStruct((P, D, S), jnp.float32),
    )(c_in)
    return jnp.transpose(c_out, (2, 0, 1))


# speedup vs baseline: 1.6354x; 1.6354x over previous
import functools

import jax
import jax.numpy as jnp
from jax import lax
from jax.experimental import pallas as pl
from jax.experimental.pallas import tpu as pltpu
from jax.experimental.pallas import tpu_sc as plsc

NC = 2
NS = 16
NW = NC * NS
L = 16

D = 64
G = 128
NBUF = 4


def _rsqrt_newton(x):
    i = lax.bitcast_convert_type(x, jnp.int32)
    i = jnp.int32(0x5F3759DF) - (i >> 1)
    y = lax.bitcast_convert_type(i, jnp.float32)
    for _ in range(3):
        y = y * (jnp.float32(1.5) - jnp.float32(0.5) * x * y * y)
    return y


def _body(ngroups, table_hbm, idx_hbm, out_hbm, idx_v, rows_v, gsem, osem):
    cid = lax.axis_index("c")
    sid = lax.axis_index("s")
    wid = sid * NC + cid
    base = wid * ngroups * G

    pltpu.sync_copy(idx_hbm.at[wid], idx_v)

    def start_gather(g, b):
        pltpu.async_copy(table_hbm.at[idx_v.at[g]], rows_v.at[b], gsem.at[b])

    def wait_gather(b):
        pltpu.make_async_copy(
            table_hbm.at[idx_v.at[0]], rows_v.at[b], gsem.at[b]
        ).wait()

    def start_out(g, b):
        pltpu.async_copy(
            rows_v.at[b], out_hbm.at[pl.ds(base + g * G, G)], osem.at[b]
        )

    def wait_out(b):
        pltpu.make_async_copy(
            rows_v.at[b], out_hbm.at[pl.ds(base, G)], osem.at[b]
        ).wait()

    RU = 8

    def compute(b):
        buf = rows_v.at[b]

        def quad(qb, carry):
            row0 = qb * RU
            for r in range(RU):
                row = row0 + r
                vs = [buf[row, pl.ds(c * L, L)] for c in range(D // L)]
                sq = [v * v for v in vs]
                ssq = (sq[0] + sq[1]) + (sq[2] + sq[3])
                s = jnp.sum(ssq)
                scale = jnp.where(
                    s > jnp.float32(1.0), _rsqrt_newton(s), jnp.float32(1.0)
                )
                sv = jnp.full((L,), scale, dtype=jnp.float32)
                for c in range(D // L):
                    buf[row, pl.ds(c * L, L)] = vs[c] * sv
            return carry

        lax.fori_loop(0, G // RU, quad, 0)

    def group(g, b, first, prefetch):
        pb = (b - 1) % NBUF
        if not first:
            wait_out(pb)
        if prefetch:
            start_gather(g + (NBUF - 1), pb)
        wait_gather(b)
        compute(b)
        start_out(g, b)

    for b in range(NBUF - 1):
        start_gather(b, b)

    for b in range(NBUF):
        group(b, b, first=(b == 0), prefetch=True)

    @pl.loop(NBUF, ngroups - NBUF, step=NBUF)
    def steady(gbase):
        for b in range(NBUF):
            group(gbase + b, b, first=False, prefetch=True)

    for b in range(NBUF):
        group(ngroups - NBUF + b, b, first=False, prefetch=(b == 0))

    wait_out(NBUF - 1)


@functools.partial(jax.jit, static_argnames=())
def kernel(token_ids, table):
    orig_shape = token_ids.shape
    B = token_ids.size
    assert B % (NW * G) == 0
    ngroups = B // (NW * G)
    assert ngroups % NBUF == 0 and ngroups >= 2 * NBUF
    idx = token_ids.reshape(NW, ngroups, G).astype(jnp.int32)

    mesh = plsc.VectorSubcoreMesh(
        core_axis_name="c", subcore_axis_name="s", num_cores=NC, num_subcores=NS
    )
    out = pl.kernel(
        functools.partial(_body, ngroups),
        out_type=jax.ShapeDtypeStruct((B, D), jnp.float32),
        mesh=mesh,
        scratch_types=[
            pltpu.VMEM((ngroups, G), jnp.int32),
            pltpu.VMEM((NBUF, G, D), jnp.float32),
            pltpu.SemaphoreType.DMA((NBUF,)),
            pltpu.SemaphoreType.DMA((NBUF,)),
        ],
        compiler_params=pltpu.CompilerParams(
            needs_layout_passes=False, use_tc_tiling_on_sc=False
        ),
    )(table, idx)
    return out.reshape(*orig_shape, D)
